# Initial kernel scaffold; baseline (speedup 1.0000x reference)
import jax
import jax.numpy as jnp
from jax.experimental import pallas as pl


def kernel(x, edge_index, edge_attr, batch, W_i, W_e, W_h, W_o, W1, b1, W2, b2):
    raise NotImplementedError("write your pallas kernel here")


# speedup vs baseline: 2.5224x; 2.5224x over previous
import functools

import jax
import jax.numpy as jnp
from jax import lax
from jax.experimental import pallas as pl
from jax.experimental.pallas import tpu as pltpu
from jax.experimental.pallas import tpu_sc as plsc

N = 10000
E = 320000
DF = 128
DE = 16
H = 128
NM = 1000
DEPTH = 3

NPAD = 10240
EPAD = 327680
MPAD = 1024
KE = 128
CH_E = EPAD // (32 * KE)
KA = 64
CH_A = NPAD // (32 * KA)
ROWS_T = NPAD // 16
MROWS_T = MPAD // 16

_mesh = plsc.VectorSubcoreMesh(core_axis_name="c", subcore_axis_name="s")
_f32 = jnp.float32


def _fill_rows(buf, val, nrows):
    v = jnp.full((16,), val, _f32)

    def body(i, carry):
        for l in range(8):
            buf[i, pl.ds(l * 16, 16)] = v
        return carry

    lax.fori_loop(0, nrows, body, 0)


@functools.partial(
    pl.kernel,
    out_type=jax.ShapeDtypeStruct((2, NPAD, H), _f32),
    mesh=_mesh,
    scratch_types=[
        pltpu.VMEM((CH_E, KE), jnp.int32),
        pltpu.VMEM((CH_E, KE), jnp.int32),
        pltpu.VMEM((KE, H), _f32),
        pltpu.VMEM((64, H), _f32),
        pltpu.VMEM_SHARED((NPAD, H), _f32),
        pltpu.SemaphoreType.DMA,
    ],
)
def _sc_gather_scatter(h_hbm, src_hbm, dst_hbm, out_hbm,
                       sidx, didx, rows, ztile, accum, sem):
    c = lax.axis_index("c")
    s = lax.axis_index("s")
    wid = c * 16 + s
    _fill_rows(ztile, 0.0, 64)

    def zbody(i, carry):
        pltpu.sync_copy(ztile, accum.at[pl.ds(s * ROWS_T + i * 64, 64)])
        return carry

    lax.fori_loop(0, ROWS_T // 64, zbody, 0)
    pltpu.sync_copy(src_hbm.at[pl.ds(wid * CH_E, CH_E)], sidx)
    pltpu.sync_copy(dst_hbm.at[pl.ds(wid * CH_E, CH_E)], didx)
    plsc.subcore_barrier()

    def body(j, carry):
        pltpu.async_copy(h_hbm.at[sidx.at[j]], rows, sem).wait()
        pltpu.sync_copy(rows, accum.at[didx.at[j]], add=True)
        return carry

    lax.fori_loop(0, CH_E, body, 0)
    plsc.subcore_barrier()
    pltpu.sync_copy(accum.at[pl.ds(s * ROWS_T, ROWS_T)],
                    out_hbm.at[c, pl.ds(s * ROWS_T, ROWS_T)])


@functools.partial(
    pl.kernel,
    out_type=jax.ShapeDtypeStruct((2, NPAD, H), _f32),
    mesh=_mesh,
    scratch_types=[
        pltpu.VMEM((CH_E, KE), jnp.int32),
        pltpu.VMEM((KE, H), _f32),
        pltpu.VMEM((64, H), _f32),
        pltpu.VMEM_SHARED((NPAD, H), _f32),
    ],
)
def _sc_scatter_linear(val_hbm, dst_hbm, out_hbm, didx, rows, ztile, accum):
    c = lax.axis_index("c")
    s = lax.axis_index("s")
    wid = c * 16 + s
    _fill_rows(ztile, 0.0, 64)

    def zbody(i, carry):
        pltpu.sync_copy(ztile, accum.at[pl.ds(s * ROWS_T + i * 64, 64)])
        return carry

    lax.fori_loop(0, ROWS_T // 64, zbody, 0)
    pltpu.sync_copy(dst_hbm.at[pl.ds(wid * CH_E, CH_E)], didx)
    plsc.subcore_barrier()

    def body(j, carry):
        pltpu.sync_copy(val_hbm.at[pl.ds((wid * CH_E + j) * KE, KE)], rows)
        pltpu.sync_copy(rows, accum.at[didx.at[j]], add=True)
        return carry

    lax.fori_loop(0, CH_E, body, 0)
    plsc.subcore_barrier()
    pltpu.sync_copy(accum.at[pl.ds(s * ROWS_T, ROWS_T)],
                    out_hbm.at[c, pl.ds(s * ROWS_T, ROWS_T)])


@functools.partial(
    pl.kernel,
    out_type=[jax.ShapeDtypeStruct((2, MPAD, H), _f32),
              jax.ShapeDtypeStruct((2, MPAD, H), _f32)],
    mesh=_mesh,
    scratch_types=[
        pltpu.VMEM((NPAD // KA, KA), jnp.int32),
        pltpu.VMEM((KA, H), _f32),
        pltpu.VMEM((KA, H), _f32),
        pltpu.VMEM((MROWS_T, H), _f32),
        pltpu.VMEM_SHARED((MPAD, H), _f32),
        pltpu.VMEM_SHARED((MPAD, H), _f32),
    ],
)
def _sc_readout(a_hbm, b_hbm, mol_hbm, cnt_hbm,
                bidx, rows, ones, ztile, macc, cacc):
    c = lax.axis_index("c")
    s = lax.axis_index("s")
    wid = c * 16 + s
    _fill_rows(ones, 1.0, KA)
    _fill_rows(ztile, 0.0, MROWS_T)
    pltpu.sync_copy(ztile, macc.at[pl.ds(s * MROWS_T, MROWS_T)])
    pltpu.sync_copy(ztile, cacc.at[pl.ds(s * MROWS_T, MROWS_T)])
    pltpu.sync_copy(b_hbm, bidx)
    plsc.subcore_barrier()

    def body(j, carry):
        pltpu.sync_copy(a_hbm.at[pl.ds((wid * CH_A + j) * KA, KA)], rows)
        pltpu.sync_copy(rows, macc.at[bidx.at[wid * CH_A + j]], add=True)
        pltpu.sync_copy(ones, cacc.at[bidx.at[wid * CH_A + j]], add=True)
        return carry

    lax.fori_loop(0, CH_A, body, 0)
    plsc.subcore_barrier()
    pltpu.sync_copy(macc.at[pl.ds(s * MROWS_T, MROWS_T)],
                    mol_hbm.at[c, pl.ds(s * MROWS_T, MROWS_T)])
    pltpu.sync_copy(cacc.at[pl.ds(s * MROWS_T, MROWS_T)],
                    cnt_hbm.at[c, pl.ds(s * MROWS_T, MROWS_T)])


def _dot(a, b):
    return jnp.dot(a, b, preferred_element_type=_f32)


def _emb_body(ea_ref, we_ref, o_ref):
    o_ref[...] = jnp.maximum(_dot(ea_ref[...], we_ref[...]), 0.0)


def _edge_embed(eap, W_e):
    bm = 5120
    return pl.pallas_call(
        _emb_body,
        grid=(EPAD // bm,),
        in_specs=[pl.BlockSpec((bm, DE), lambda i: (i, 0)),
                  pl.BlockSpec((DE, H), lambda i: (0, 0))],
        out_specs=pl.BlockSpec((bm, H), lambda i: (i, 0)),
        out_shape=jax.ShapeDtypeStruct((EPAD, H), _f32),
    )(eap, W_e)


def _base_body(x_ref, wi_ref, e0_ref, e1_ref, wh_ref, h0_ref, base_ref):
    h0 = jnp.maximum(_dot(x_ref[...], wi_ref[...]), 0.0)
    h0_ref[...] = h0
    base_ref[...] = h0 + _dot(e0_ref[...] + e1_ref[...], wh_ref[...])


def _make_base(xp, W_i, e0, e1, W_h):
    bm = 1024
    return pl.pallas_call(
        _base_body,
        grid=(NPAD // bm,),
        in_specs=[pl.BlockSpec((bm, DF), lambda i: (i, 0)),
                  pl.BlockSpec((DF, H), lambda i: (0, 0)),
                  pl.BlockSpec((bm, H), lambda i: (i, 0)),
                  pl.BlockSpec((bm, H), lambda i: (i, 0)),
                  pl.BlockSpec((H, H), lambda i: (0, 0))],
        out_specs=[pl.BlockSpec((bm, H), lambda i: (i, 0)),
                   pl.BlockSpec((bm, H), lambda i: (i, 0))],
        out_shape=[jax.ShapeDtypeStruct((NPAD, H), _f32),
                   jax.ShapeDtypeStruct((NPAD, H), _f32)],
    )(xp, W_i, e0, e1, W_h)


def _upd_body(a0_ref, a1_ref, base_ref, wh_ref, h_ref):
    h_ref[...] = jnp.maximum(
        base_ref[...] + _dot(a0_ref[...] + a1_ref[...], wh_ref[...]), 0.0)


def _update_h(a0, a1, base, W_h):
    bm = 1024
    return pl.pallas_call(
        _upd_body,
        grid=(NPAD // bm,),
        in_specs=[pl.BlockSpec((bm, H), lambda i: (i, 0)),
                  pl.BlockSpec((bm, H), lambda i: (i, 0)),
                  pl.BlockSpec((bm, H), lambda i: (i, 0)),
                  pl.BlockSpec((H, H), lambda i: (0, 0))],
        out_specs=pl.BlockSpec((bm, H), lambda i: (i, 0)),
        out_shape=jax.ShapeDtypeStruct((NPAD, H), _f32),
    )(a0, a1, base, W_h)


def _atom_body(x_ref, h_ref, wx_ref, wh_ref, a_ref):
    a_ref[...] = jnp.maximum(
        _dot(x_ref[...], wx_ref[...]) + _dot(h_ref[...], wh_ref[...]), 0.0)


def _atom_out(xp, h, W_ox, W_oh):
    bm = 1024
    return pl.pallas_call(
        _atom_body,
        grid=(NPAD // bm,),
        in_specs=[pl.BlockSpec((bm, DF), lambda i: (i, 0)),
                  pl.BlockSpec((bm, H), lambda i: (i, 0)),
                  pl.BlockSpec((DF, H), lambda i: (0, 0)),
                  pl.BlockSpec((H, H), lambda i: (0, 0))],
        out_specs=pl.BlockSpec((bm, H), lambda i: (i, 0)),
        out_shape=jax.ShapeDtypeStruct((NPAD, H), _f32),
    )(xp, h, W_ox, W_oh)


def _head_body(m0_ref, m1_ref, c0_ref, c1_ref, w1_ref, b1_ref, w2_ref,
               b2_ref, o_ref):
    cnt = jnp.maximum(c0_ref[...] + c1_ref[...], 1.0)
    mol = (m0_ref[...] + m1_ref[...]) / cnt
    hdn = jnp.maximum(_dot(mol, w1_ref[...]) + b1_ref[...], 0.0)
    o_ref[...] = _dot(hdn, w2_ref[...]) + b2_ref[...]


def _head(m0, m1, c0, c1, W1, b1, W2, b2):
    return pl.pallas_call(
        _head_body,
        out_shape=jax.ShapeDtypeStruct((MPAD, 1), _f32),
    )(m0, m1, c0, c1, W1, b1, W2, b2)


def kernel(x, edge_index, edge_attr, batch, W_i, W_e, W_h, W_o, W1, b1, W2, b2):
    xp = jnp.zeros((NPAD, DF), _f32).at[:N].set(x)
    src = jnp.concatenate(
        [edge_index[0], jnp.zeros((EPAD - E,), jnp.int32)]).reshape(EPAD // KE, KE)
    dst = jnp.concatenate(
        [edge_index[1], jnp.full((EPAD - E,), N, jnp.int32)]).reshape(EPAD // KE, KE)
    eap = jnp.zeros((EPAD, DE), _f32).at[:E].set(edge_attr)
    bp = jnp.concatenate(
        [batch, jnp.full((NPAD - N,), NM, jnp.int32)]).reshape(NPAD // KA, KA)

    e_emb = _edge_embed(eap, W_e)
    eagg = _sc_scatter_linear(e_emb, dst)
    h0, base = _make_base(xp, W_i, eagg[0], eagg[1], W_h)
    h = h0
    for _ in range(DEPTH):
        agg = _sc_gather_scatter(h, src, dst)
        h = _update_h(agg[0], agg[1], base, W_h)
    a = _atom_out(xp, h, W_o[:DF], W_o[DF:])
    mol2, cnt2 = _sc_readout(a, bp)
    out = _head(mol2[0], mol2[1], cnt2[0], cnt2[1],
                W1, b1.reshape(1, H), W2, b2.reshape(1, 1))
    return out[:NM]


# speedup vs baseline: 2.8252x; 1.1200x over previous
import functools

import jax
import jax.numpy as jnp
from jax import lax
from jax.experimental import pallas as pl
from jax.experimental.pallas import tpu as pltpu
from jax.experimental.pallas import tpu_sc as plsc

N = 10000
E = 320000
DF = 128
DE = 16
H = 128
NM = 1000
DEPTH = 3

NPAD = 10240
EPAD = 327680
MPAD = 1024
KE = 64
CH_E = EPAD // (32 * KE)
CHH = CH_E // 2
KA = 64
CH_A = NPAD // (32 * KA)
ROWS_T = NPAD // 16
MROWS_T = MPAD // 16

_mesh = plsc.VectorSubcoreMesh(core_axis_name="c", subcore_axis_name="s")
_f32 = jnp.float32


def _fill_rows(buf, val, nrows):
    v = jnp.full((16,), val, _f32)

    def body(i, carry):
        for l in range(8):
            buf[i, pl.ds(l * 16, 16)] = v
        return carry

    lax.fori_loop(0, nrows, body, 0)


NBUF = 2


@functools.partial(
    pl.kernel,
    out_type=jax.ShapeDtypeStruct((2, NPAD, H), _f32),
    mesh=_mesh,
    scratch_types=[
        pltpu.VMEM((2 * CHH, KE), jnp.int32),
        pltpu.VMEM((NBUF, KE, H), _f32),
        pltpu.VMEM_SHARED((NPAD, H), _f32),
        pltpu.SemaphoreType.DMA,
        pltpu.SemaphoreType.DMA,
    ],
)
def _sc_gather_scatter(h_hbm, src_hbm, dst_hbm, out_hbm,
                       idx, rows, accum, *sems):
    c = lax.axis_index("c")
    s = lax.axis_index("s")
    wid = c * 16 + s
    _fill_rows(rows.at[0], 0.0, KE)

    def zbody(i, carry):
        pltpu.sync_copy(rows.at[0], accum.at[pl.ds(s * ROWS_T + i * KE, KE)])
        return carry

    lax.fori_loop(0, ROWS_T // KE, zbody, 0)
    plsc.subcore_barrier()

    for half in range(CH_E // CHH):
        base = wid * CH_E + half * CHH
        pltpu.sync_copy(src_hbm.at[pl.ds(base, CHH)], idx.at[pl.ds(0, CHH)])
        pltpu.sync_copy(dst_hbm.at[pl.ds(base, CHH)], idx.at[pl.ds(CHH, CHH)])

        for b in range(NBUF):
            pltpu.async_copy(h_hbm.at[idx.at[b]], rows.at[b], sems[b])

        def body(g, carry):
            for b in range(NBUF):
                j = g * NBUF + b
                pltpu.make_async_copy(h_hbm.at[idx.at[j]], rows.at[b],
                                      sems[b]).wait()
                pltpu.sync_copy(rows.at[b], accum.at[idx.at[CHH + j]],
                                add=True)
                pltpu.async_copy(h_hbm.at[idx.at[j + NBUF]], rows.at[b],
                                 sems[b])
            return carry

        lax.fori_loop(0, CHH // NBUF - 1, body, 0)
        for b in range(NBUF):
            j = CHH - NBUF + b
            pltpu.make_async_copy(h_hbm.at[idx.at[j]], rows.at[b],
                                  sems[b]).wait()
            pltpu.sync_copy(rows.at[b], accum.at[idx.at[CHH + j]], add=True)
    plsc.subcore_barrier()
    pltpu.sync_copy(accum.at[pl.ds(s * ROWS_T, ROWS_T)],
                    out_hbm.at[c, pl.ds(s * ROWS_T, ROWS_T)])


@functools.partial(
    pl.kernel,
    out_type=jax.ShapeDtypeStruct((2, NPAD, H), _f32),
    mesh=_mesh,
    scratch_types=[
        pltpu.VMEM((CH_E, KE), jnp.int32),
        pltpu.VMEM((NBUF, KE, H), _f32),
        pltpu.VMEM_SHARED((NPAD, H), _f32),
        pltpu.SemaphoreType.DMA,
        pltpu.SemaphoreType.DMA,
    ],
)
def _sc_scatter_linear(val_hbm, dst_hbm, out_hbm, didx, rows, accum,
                       *sems):
    c = lax.axis_index("c")
    s = lax.axis_index("s")
    wid = c * 16 + s
    _fill_rows(rows.at[0], 0.0, KE)

    def zbody(i, carry):
        pltpu.sync_copy(rows.at[0], accum.at[pl.ds(s * ROWS_T + i * KE, KE)])
        return carry

    lax.fori_loop(0, ROWS_T // KE, zbody, 0)
    pltpu.sync_copy(dst_hbm.at[pl.ds(wid * CH_E, CH_E)], didx)
    plsc.subcore_barrier()

    for b in range(NBUF):
        pltpu.async_copy(val_hbm.at[pl.ds((wid * CH_E + b) * KE, KE)],
                         rows.at[b], sems[b])

    def body(g, carry):
        for b in range(NBUF):
            j = g * NBUF + b
            pltpu.make_async_copy(
                val_hbm.at[pl.ds((wid * CH_E + j) * KE, KE)], rows.at[b],
                sems[b]).wait()
            pltpu.sync_copy(rows.at[b], accum.at[didx.at[j]], add=True)
            pltpu.async_copy(
                val_hbm.at[pl.ds((wid * CH_E + j + NBUF) * KE, KE)],
                rows.at[b], sems[b])
        return carry

    lax.fori_loop(0, CH_E // NBUF - 1, body, 0)
    for b in range(NBUF):
        j = CH_E - NBUF + b
        pltpu.make_async_copy(
            val_hbm.at[pl.ds((wid * CH_E + j) * KE, KE)], rows.at[b],
            sems[b]).wait()
        pltpu.sync_copy(rows.at[b], accum.at[didx.at[j]], add=True)
    plsc.subcore_barrier()
    pltpu.sync_copy(accum.at[pl.ds(s * ROWS_T, ROWS_T)],
                    out_hbm.at[c, pl.ds(s * ROWS_T, ROWS_T)])


@functools.partial(
    pl.kernel,
    out_type=[jax.ShapeDtypeStruct((2, MPAD, H), _f32),
              jax.ShapeDtypeStruct((2, MPAD, H), _f32)],
    mesh=_mesh,
    scratch_types=[
        pltpu.VMEM((NPAD // KA, KA), jnp.int32),
        pltpu.VMEM((KA, H), _f32),
        pltpu.VMEM((KA, H), _f32),
        pltpu.VMEM((MROWS_T, H), _f32),
        pltpu.VMEM_SHARED((MPAD, H), _f32),
        pltpu.VMEM_SHARED((MPAD, H), _f32),
    ],
)
def _sc_readout(a_hbm, b_hbm, mol_hbm, cnt_hbm,
                bidx, rows, ones, ztile, macc, cacc):
    c = lax.axis_index("c")
    s = lax.axis_index("s")
    wid = c * 16 + s
    _fill_rows(ones, 1.0, KA)
    _fill_rows(ztile, 0.0, MROWS_T)
    pltpu.sync_copy(ztile, macc.at[pl.ds(s * MROWS_T, MROWS_T)])
    pltpu.sync_copy(ztile, cacc.at[pl.ds(s * MROWS_T, MROWS_T)])
    pltpu.sync_copy(b_hbm, bidx)
    plsc.subcore_barrier()

    def body(j, carry):
        pltpu.sync_copy(a_hbm.at[pl.ds((wid * CH_A + j) * KA, KA)], rows)
        pltpu.sync_copy(rows, macc.at[bidx.at[wid * CH_A + j]], add=True)
        pltpu.sync_copy(ones, cacc.at[bidx.at[wid * CH_A + j]], add=True)
        return carry

    lax.fori_loop(0, CH_A, body, 0)
    plsc.subcore_barrier()
    pltpu.sync_copy(macc.at[pl.ds(s * MROWS_T, MROWS_T)],
                    mol_hbm.at[c, pl.ds(s * MROWS_T, MROWS_T)])
    pltpu.sync_copy(cacc.at[pl.ds(s * MROWS_T, MROWS_T)],
                    cnt_hbm.at[c, pl.ds(s * MROWS_T, MROWS_T)])


def _dot(a, b):
    return jnp.dot(a, b, preferred_element_type=_f32)


def _emb_body(ea_ref, we_ref, o_ref):
    o_ref[...] = jnp.maximum(_dot(ea_ref[...], we_ref[...]), 0.0)


def _edge_embed(eap, W_e):
    bm = 5120
    return pl.pallas_call(
        _emb_body,
        grid=(EPAD // bm,),
        in_specs=[pl.BlockSpec((bm, DE), lambda i: (i, 0)),
                  pl.BlockSpec((DE, H), lambda i: (0, 0))],
        out_specs=pl.BlockSpec((bm, H), lambda i: (i, 0)),
        out_shape=jax.ShapeDtypeStruct((EPAD, H), _f32),
    )(eap, W_e)


def _base_body(x_ref, wi_ref, e0_ref, e1_ref, wh_ref, h0_ref, base_ref):
    h0 = jnp.maximum(_dot(x_ref[...], wi_ref[...]), 0.0)
    h0_ref[...] = h0
    base_ref[...] = h0 + _dot(e0_ref[...] + e1_ref[...], wh_ref[...])


def _make_base(xp, W_i, e0, e1, W_h):
    bm = 1024
    return pl.pallas_call(
        _base_body,
        grid=(NPAD // bm,),
        in_specs=[pl.BlockSpec((bm, DF), lambda i: (i, 0)),
                  pl.BlockSpec((DF, H), lambda i: (0, 0)),
                  pl.BlockSpec((bm, H), lambda i: (i, 0)),
                  pl.BlockSpec((bm, H), lambda i: (i, 0)),
                  pl.BlockSpec((H, H), lambda i: (0, 0))],
        out_specs=[pl.BlockSpec((bm, H), lambda i: (i, 0)),
                   pl.BlockSpec((bm, H), lambda i: (i, 0))],
        out_shape=[jax.ShapeDtypeStruct((NPAD, H), _f32),
                   jax.ShapeDtypeStruct((NPAD, H), _f32)],
    )(xp, W_i, e0, e1, W_h)


def _upd_body(a0_ref, a1_ref, base_ref, wh_ref, h_ref):
    h_ref[...] = jnp.maximum(
        base_ref[...] + _dot(a0_ref[...] + a1_ref[...], wh_ref[...]), 0.0)


def _update_h(a0, a1, base, W_h):
    bm = 1024
    return pl.pallas_call(
        _upd_body,
        grid=(NPAD // bm,),
        in_specs=[pl.BlockSpec((bm, H), lambda i: (i, 0)),
                  pl.BlockSpec((bm, H), lambda i: (i, 0)),
                  pl.BlockSpec((bm, H), lambda i: (i, 0)),
                  pl.BlockSpec((H, H), lambda i: (0, 0))],
        out_specs=pl.BlockSpec((bm, H), lambda i: (i, 0)),
        out_shape=jax.ShapeDtypeStruct((NPAD, H), _f32),
    )(a0, a1, base, W_h)


def _atom_body(x_ref, h_ref, wx_ref, wh_ref, a_ref):
    a_ref[...] = jnp.maximum(
        _dot(x_ref[...], wx_ref[...]) + _dot(h_ref[...], wh_ref[...]), 0.0)


def _atom_out(xp, h, W_ox, W_oh):
    bm = 1024
    return pl.pallas_call(
        _atom_body,
        grid=(NPAD // bm,),
        in_specs=[pl.BlockSpec((bm, DF), lambda i: (i, 0)),
                  pl.BlockSpec((bm, H), lambda i: (i, 0)),
                  pl.BlockSpec((DF, H), lambda i: (0, 0)),
                  pl.BlockSpec((H, H), lambda i: (0, 0))],
        out_specs=pl.BlockSpec((bm, H), lambda i: (i, 0)),
        out_shape=jax.ShapeDtypeStruct((NPAD, H), _f32),
    )(xp, h, W_ox, W_oh)


def _head_body(m0_ref, m1_ref, c0_ref, c1_ref, w1_ref, b1_ref, w2_ref,
               b2_ref, o_ref):
    cnt = jnp.maximum(c0_ref[...] + c1_ref[...], 1.0)
    mol = (m0_ref[...] + m1_ref[...]) / cnt
    hdn = jnp.maximum(_dot(mol, w1_ref[...]) + b1_ref[...], 0.0)
    o_ref[...] = _dot(hdn, w2_ref[...]) + b2_ref[...]


def _head(m0, m1, c0, c1, W1, b1, W2, b2):
    return pl.pallas_call(
        _head_body,
        out_shape=jax.ShapeDtypeStruct((MPAD, 1), _f32),
    )(m0, m1, c0, c1, W1, b1, W2, b2)


def kernel(x, edge_index, edge_attr, batch, W_i, W_e, W_h, W_o, W1, b1, W2, b2):
    xp = jnp.zeros((NPAD, DF), _f32).at[:N].set(x)
    src = jnp.concatenate(
        [edge_index[0], jnp.zeros((EPAD - E,), jnp.int32)]).reshape(EPAD // KE, KE)
    dst = jnp.concatenate(
        [edge_index[1], jnp.full((EPAD - E,), N, jnp.int32)]).reshape(EPAD // KE, KE)
    eap = jnp.zeros((EPAD, DE), _f32).at[:E].set(edge_attr)
    bp = jnp.concatenate(
        [batch, jnp.full((NPAD - N,), NM, jnp.int32)]).reshape(NPAD // KA, KA)

    e_emb = _edge_embed(eap, W_e)
    eagg = _sc_scatter_linear(e_emb, dst)
    h0, base = _make_base(xp, W_i, eagg[0], eagg[1], W_h)
    h = h0
    for _ in range(DEPTH):
        agg = _sc_gather_scatter(h, src, dst)
        h = _update_h(agg[0], agg[1], base, W_h)
    a = _atom_out(xp, h, W_o[:DF], W_o[DF:])
    mol2, cnt2 = _sc_readout(a, bp)
    out = _head(mol2[0], mol2[1], cnt2[0], cnt2[1],
                W1, b1.reshape(1, H), W2, b2.reshape(1, 1))
    return out[:NM]
